# Initial kernel scaffold; baseline (speedup 1.0000x reference)
import jax
import jax.numpy as jnp
from jax.experimental import pallas as pl


def kernel(state_variable, weights, edge_index):
    raise NotImplementedError("write your pallas kernel here")


# speedup vs baseline: 4.6609x; 4.6609x over previous
import functools

import jax
import jax.numpy as jnp
from jax import lax
from jax.experimental import pallas as pl
from jax.experimental.pallas import tpu as pltpu
from jax.experimental.pallas import tpu_sc as plsc

N = 10000
E = 320000
D = 128

NC = 2
NS = 16
NW = NC * NS

E_PER_W = E // NW
CHUNK = 80
N_CHUNKS = E_PER_W // CHUNK
NPAD = 10240
ROWS_PER_TILE = NPAD // NS
LANES = 16
DL = D // LANES


def _sc_scatter(state_hbm, w_hbm, row_hbm, col_hbm, zeros_hbm, out_hbm,
                accum, idx_row, idx_col, wbuf, wexp, rows_a, rows_b, res,
                sem_a, sem_b):
    c = lax.axis_index("c")
    s = lax.axis_index("s")
    wid = s * NC + c
    base0 = wid * E_PER_W

    zsl = pl.ds(s * ROWS_PER_TILE, ROWS_PER_TILE)
    pltpu.sync_copy(zeros_hbm.at[zsl], accum.at[zsl])
    plsc.subcore_barrier()

    def step(g, _):
        base = base0 + g * CHUNK
        pltpu.sync_copy(row_hbm.at[pl.ds(base, CHUNK)], idx_row)
        pltpu.sync_copy(col_hbm.at[pl.ds(base, CHUNK)], idx_col)
        pltpu.sync_copy(w_hbm.at[pl.ds(base, CHUNK)], wbuf)
        cp_a = pltpu.async_copy(state_hbm.at[idx_col], rows_a, sem_a)
        cp_b = pltpu.async_copy(state_hbm.at[idx_row], rows_b, sem_b)
        cp_a.wait()
        cp_b.wait()

        def wexpand(k, _):
            wv = wbuf[pl.ds(k * LANES, LANES)]
            for e16 in range(LANES):
                wexp[k * LANES + e16, :] = jnp.full((LANES,), wv[e16], jnp.float32)
            return _
        lax.fori_loop(0, CHUNK // LANES, wexpand, None)

        def edge(e, _):
            wv = wexp[e, :]
            for j in range(DL):
                sl = pl.ds(j * LANES, LANES)
                res[e, sl] = wv * (rows_a[e, sl] - rows_b[e, sl])
            return _
        lax.fori_loop(0, CHUNK, edge, None)
        pltpu.sync_copy(res, accum.at[idx_row], add=True)
        return _
    lax.fori_loop(0, N_CHUNKS, step, None)

    plsc.subcore_barrier()
    sl = pl.ds(s * ROWS_PER_TILE, ROWS_PER_TILE)
    pltpu.sync_copy(accum.at[sl], out_hbm.at[c, sl])


def _tc_add(p_ref, o_ref):
    o_ref[...] = p_ref[0] + p_ref[1]


@jax.jit
def kernel(state_variable, weights, edge_index):
    row = edge_index[0]
    col = edge_index[1]
    mesh = plsc.VectorSubcoreMesh(core_axis_name="c", subcore_axis_name="s")
    partial = pl.kernel(
        _sc_scatter,
        mesh=mesh,
        out_type=jax.ShapeDtypeStruct((NC, NPAD, D), jnp.float32),
        scratch_types=[
            pltpu.VMEM_SHARED((NPAD, D), jnp.float32),
            pltpu.VMEM((CHUNK,), jnp.int32),
            pltpu.VMEM((CHUNK,), jnp.int32),
            pltpu.VMEM((CHUNK,), jnp.float32),
            pltpu.VMEM((CHUNK, LANES), jnp.float32),
            pltpu.VMEM((CHUNK, D), jnp.float32),
            pltpu.VMEM((CHUNK, D), jnp.float32),
            pltpu.VMEM((CHUNK, D), jnp.float32),
            pltpu.SemaphoreType.DMA,
            pltpu.SemaphoreType.DMA,
        ],
    )(state_variable, weights, row, col, jnp.zeros((NPAD, D), jnp.float32))

    nblk = 10
    return pl.pallas_call(
        _tc_add,
        grid=(nblk,),
        in_specs=[pl.BlockSpec((NC, N // nblk, D), lambda i: (0, i, 0))],
        out_specs=pl.BlockSpec((N // nblk, D), lambda i: (i, 0)),
        out_shape=jax.ShapeDtypeStruct((N, D), jnp.float32),
    )(partial)


# speedup vs baseline: 5.5802x; 1.1972x over previous
import jax
import jax.numpy as jnp
from jax import lax
from jax.experimental import pallas as pl
from jax.experimental.pallas import tpu as pltpu
from jax.experimental.pallas import tpu_sc as plsc

N = 10000
E = 320000
D = 128

NC = 2
NS = 16
NW = NC * NS

E_PER_W = E // NW
CHUNK = 80
N_CHUNKS = E_PER_W // CHUNK
NPAD = 10240
ROWS_PER_TILE = NPAD // NS
LANES = 16
DL = D // LANES


def _sc_scatter(state_hbm, w_hbm, row_hbm, col_hbm, zeros_hbm,
                out_hbm, deg_hbm,
                accum, degacc, idx_row, idx_col, wbuf, wexp, rows_a, res,
                sem_a):
    c = lax.axis_index("c")
    s = lax.axis_index("s")
    wid = s * NC + c
    base0 = wid * E_PER_W

    zsl = pl.ds(s * ROWS_PER_TILE, ROWS_PER_TILE)
    pltpu.sync_copy(zeros_hbm.at[zsl], accum.at[zsl])

    def dzero(k, _):
        degacc[pl.ds(k * LANES, LANES)] = jnp.zeros((LANES,), jnp.float32)
        return _
    lax.fori_loop(0, NPAD // LANES, dzero, None)
    plsc.subcore_barrier()

    def step(g, _):
        base = base0 + g * CHUNK
        pltpu.sync_copy(row_hbm.at[pl.ds(base, CHUNK)], idx_row)
        pltpu.sync_copy(col_hbm.at[pl.ds(base, CHUNK)], idx_col)
        pltpu.sync_copy(w_hbm.at[pl.ds(base, CHUNK)], wbuf)
        cp_a = pltpu.async_copy(state_hbm.at[idx_col], rows_a, sem_a)

        def wexpand(k, _):
            kl = k * LANES
            wv = wbuf[pl.ds(kl, LANES)]
            iv = idx_row[pl.ds(kl, LANES)]
            plsc.addupdate_scatter(degacc, [iv], wv)
            for e16 in range(LANES):
                wexp[kl + e16, :] = jnp.full((LANES,), wv[e16], jnp.float32)
            return _
        lax.fori_loop(0, CHUNK // LANES, wexpand, None)
        cp_a.wait()

        def edge(e, _):
            wv = wexp[e, :]
            for j in range(DL):
                sl = pl.ds(j * LANES, LANES)
                res[e, sl] = wv * rows_a[e, sl]
            return _
        lax.fori_loop(0, CHUNK, edge, None)
        pltpu.sync_copy(res, accum.at[idx_row], add=True)
        return _
    lax.fori_loop(0, N_CHUNKS, step, None)

    plsc.subcore_barrier()
    sl = pl.ds(s * ROWS_PER_TILE, ROWS_PER_TILE)
    pltpu.sync_copy(accum.at[sl], out_hbm.at[c, sl])
    pltpu.sync_copy(degacc, deg_hbm.at[wid])


def _tc_combine(p_ref, deg_ref, state_ref, o_ref):
    deg = jnp.sum(deg_ref[...], axis=0)
    o_ref[...] = p_ref[0] + p_ref[1] - deg[:, None] * state_ref[...]


@jax.jit
def kernel(state_variable, weights, edge_index):
    row = edge_index[0]
    col = edge_index[1]
    mesh = plsc.VectorSubcoreMesh(core_axis_name="c", subcore_axis_name="s")
    partial, degs = pl.kernel(
        _sc_scatter,
        mesh=mesh,
        compiler_params=pltpu.CompilerParams(needs_layout_passes=False),
        out_type=(
            jax.ShapeDtypeStruct((NC, NPAD, D), jnp.float32),
            jax.ShapeDtypeStruct((NW, NPAD), jnp.float32),
        ),
        scratch_types=[
            pltpu.VMEM_SHARED((NPAD, D), jnp.float32),
            pltpu.VMEM((NPAD,), jnp.float32),
            pltpu.VMEM((CHUNK,), jnp.int32),
            pltpu.VMEM((CHUNK,), jnp.int32),
            pltpu.VMEM((CHUNK,), jnp.float32),
            pltpu.VMEM((CHUNK, LANES), jnp.float32),
            pltpu.VMEM((CHUNK, D), jnp.float32),
            pltpu.VMEM((CHUNK, D), jnp.float32),
            pltpu.SemaphoreType.DMA,
        ],
    )(state_variable, weights, row, col, jnp.zeros((NPAD, D), jnp.float32))

    nblk = 10
    blk = NPAD // nblk
    return pl.pallas_call(
        _tc_combine,
        grid=(nblk,),
        in_specs=[
            pl.BlockSpec((NC, blk, D), lambda i: (0, i, 0)),
            pl.BlockSpec((NW, blk), lambda i: (0, i)),
            pl.BlockSpec((blk, D), lambda i: (i, 0)),
        ],
        out_specs=pl.BlockSpec((blk, D), lambda i: (i, 0)),
        out_shape=jax.ShapeDtypeStruct((N, D), jnp.float32),
    )(partial, degs, state_variable)
